# baseline (speedup 1.0000x reference)
import functools

import jax
import jax.numpy as jnp
from jax import lax
from jax.experimental import pallas as pl
from jax.experimental.pallas import tpu as pltpu
from jax.experimental.pallas import tpu_sc as plsc

_B = 16384
_D = 32
_CHUNK = 512
_CAP_U = 999936
_CAP_M = 99968
_STRIPE_U = 31360
_STRIPE_M = 3200


def _sc_side(idx, ta, tb, cap, stripe, nchunks):
    info = plsc.get_sparse_core_info()
    mesh = plsc.VectorSubcoreMesh(core_axis_name="c", subcore_axis_name="s")

    @functools.partial(
        pl.kernel,
        mesh=mesh,
        compiler_params=pltpu.CompilerParams(needs_layout_passes=False),
        out_type=jax.ShapeDtypeStruct((_B + 16, 4 * _D), jnp.float32),
        scratch_types=[
            pltpu.VMEM((_B,), jnp.int32),
            pltpu.VMEM((_B,), jnp.int32),
            pltpu.VMEM((_B,), jnp.int32),
            pltpu.VMEM((_B,), jnp.int32),
            pltpu.VMEM((_B,), jnp.int32),
            pltpu.VMEM((8, _CHUNK), jnp.float32),
            pltpu.VMEM((8, _CHUNK), jnp.float32),
            pltpu.VMEM((8, _CHUNK), jnp.float32),
            pltpu.VMEM((8, _CHUNK), jnp.float32),
            pltpu.VMEM((8, _CHUNK), jnp.float32),
            pltpu.VMEM((8, _CHUNK), jnp.float32),
            pltpu.VMEM((8, _CHUNK), jnp.float32),
            pltpu.VMEM((8, _CHUNK), jnp.float32),
            pltpu.VMEM((16, 4 * _D), jnp.float32),
            pltpu.SemaphoreType.DMA,
            pltpu.SemaphoreType.DMA,
        ],
    )
    def k(idx_h, ta_h, tb_h, out_h,
          all_idx, lidx, lpos, ccol, cpos,
          a0, a1, a2, a3, b0, bb1, bb2, bb3, staging, sem, sem2):
        cha = (a0, a1, a2, a3)
        chb = (b0, bb1, bb2, bb3)
        wid = lax.axis_index("s") * info.num_cores + lax.axis_index("c")
        t0 = wid * stripe
        pltpu.sync_copy(idx_h, all_idx)
        iota = lax.iota(jnp.int32, 16)

        def scan(i, n):
            v = all_idx[pl.ds(i * 16, 16)]
            pos = iota + i * 16
            msk = (v >= t0) & (v < t0 + stripe)
            dst = n + plsc.cumsum(msk.astype(jnp.int32)) - 1
            plsc.store_scatter(lidx, [dst], v, mask=msk)
            plsc.store_scatter(lpos, [dst], pos, mask=msk)
            return n + jnp.sum(msk.astype(jnp.int32))

        n = lax.fori_loop(0, _B // 16, scan, 0)

        def chunk(c, carry):
            s_c = pl.multiple_of(jnp.minimum(t0 + c * _CHUNK, cap - _CHUNK), 128)
            sel_lo = t0 + c * _CHUNK
            sel_hi = jnp.minimum(jnp.minimum(t0 + (c + 1) * _CHUNK, t0 + stripe),
                                 cap)
            copies = []
            for tr in range(_D // 8):
                copies.append(pltpu.async_copy(
                    ta_h.at[pl.ds(tr * 8, 8), pl.ds(s_c, _CHUNK)],
                    cha[tr], sem2))
                copies.append(pltpu.async_copy(
                    tb_h.at[pl.ds(tr * 8, 8), pl.ds(s_c, _CHUNK)],
                    chb[tr], sem2))

            def subsel(g, m):
                v = lidx[pl.ds(g * 16, 16)]
                p = lpos[pl.ds(g * 16, 16)]
                msk = (v >= sel_lo) & (v < sel_hi)
                dst = m + plsc.cumsum(msk.astype(jnp.int32)) - 1
                plsc.store_scatter(ccol, [dst], v - s_c, mask=msk)
                plsc.store_scatter(cpos, [dst], p, mask=msk)
                return m + jnp.sum(msk.astype(jnp.int32))

            m = lax.fori_loop(0, (n + 15) // 16, subsel, 0)
            for cp in copies:
                cp.wait()

            def grp(g, carry2):
                cv = ccol[pl.ds(g * 16, 16)]
                pv = cpos[pl.ds(g * 16, 16)]
                valid = (iota + g * 16) < m
                cv = jnp.where(valid, cv, 0)
                pv = jnp.where(valid, pv, _B)
                for d in range(_D):
                    sr = jnp.full((16,), d % 8, jnp.int32)
                    sd = jnp.full((16,), d, jnp.int32)
                    va = plsc.load_gather(cha[d // 8], [sr, cv])
                    plsc.store_scatter(staging, [iota, sd], va)
                    vb = plsc.load_gather(chb[d // 8], [sr, cv])
                    plsc.store_scatter(staging, [iota, sd + _D], vb)
                pltpu.async_copy(staging, out_h.at[pv], sem).wait()
                return carry2

            lax.fori_loop(0, (m + 15) // 16, grp, 0)
            return carry

        lax.fori_loop(0, nchunks, chunk, 0)

    return k(idx, ta, tb)


def _mlp_body(xu_r, xm_r, u_r, m_r, eu_r, em_r, fu_r, fm_r,
              w1_r, b1_r, w2_r, b2_r, w3_r, b3_r, wo_r, bo_r, out_r):
    f32 = jnp.float32
    xu = xu_r[...]
    xm = xm_r[...]
    u = u_r[...]
    mv = m_r[...]
    blk = xu.shape[0]
    dn_t = (((1,), (1,)), ((), ()))
    iota_l = lax.broadcasted_iota(jnp.int32, (blk, 128), 1)
    iota_t = lax.broadcasted_iota(jnp.int32, (_D, 128), 1)
    u_edge = iota_t < (1000001 - _CAP_U)
    m_edge = iota_t < (100001 - _CAP_M)
    ohu = (iota_l == (u - _CAP_U)).astype(f32)
    ohm = (iota_l == (mv - _CAP_M)).astype(f32)
    eu = jnp.where(u_edge, eu_r[...], 0.0)
    fu = jnp.where(u_edge, fu_r[...], 0.0)
    em = jnp.where(m_edge, em_r[...], 0.0)
    fm = jnp.where(m_edge, fm_r[...], 0.0)
    pgu = lax.dot_general(ohu, eu, dn_t, preferred_element_type=f32)
    pmu = lax.dot_general(ohu, fu, dn_t, preferred_element_type=f32)
    pgm = lax.dot_general(ohm, em, dn_t, preferred_element_type=f32)
    pmm = lax.dot_general(ohm, fm, dn_t, preferred_element_type=f32)
    tail_u = u >= _CAP_U
    tail_m = mv >= _CAP_M
    gu = jnp.where(tail_u, pgu, xu[:, :_D])
    mu = jnp.where(tail_u, pmu, xu[:, _D:2 * _D])
    gm = jnp.where(tail_m, pgm, xm[:, :_D])
    mm = jnp.where(tail_m, pmm, xm[:, _D:2 * _D])

    w1 = w1_r[...]
    h = (lax.dot_general(mu, w1[:, :_D], dn_t, preferred_element_type=f32)
         + lax.dot_general(mm, w1[:, _D:], dn_t, preferred_element_type=f32)
         + b1_r[...])
    h = jnp.maximum(h, 0.0)
    h = lax.dot_general(h, w2_r[...], dn_t, preferred_element_type=f32) + b2_r[...]
    h = jnp.maximum(h, 0.0)
    h = lax.dot_general(h, w3_r[...], dn_t, preferred_element_type=f32) + b3_r[...]
    h = jnp.maximum(h, 0.0)
    gmf = gu * gm
    wo = wo_r[...]
    out = (jnp.sum(gmf * wo[:, :_D], axis=1)
           + jnp.sum(h * wo[:, _D:], axis=1)
           + bo_r[0, 0])
    out_r[...] = out


def _tc_mlp(xu, xm, u3, m3, tgu, tgm, tmu, tmm,
            W1, b1, W2, b2, W3, b3, Wo, bo):
    blk = 2048
    grid = _B // blk

    def full(shape):
        return pl.BlockSpec(shape, lambda i: (0,) * len(shape))

    return pl.pallas_call(
        _mlp_body,
        grid=(grid,),
        in_specs=[
            pl.BlockSpec((blk, 4 * _D), lambda i: (i, 0)),
            pl.BlockSpec((blk, 4 * _D), lambda i: (i, 0)),
            pl.BlockSpec((blk, 1), lambda i: (i, 0)),
            pl.BlockSpec((blk, 1), lambda i: (i, 0)),
            pl.BlockSpec((_D, 128), lambda i: (0, _CAP_U // 128)),
            pl.BlockSpec((_D, 128), lambda i: (0, _CAP_M // 128)),
            pl.BlockSpec((_D, 128), lambda i: (0, _CAP_U // 128)),
            pl.BlockSpec((_D, 128), lambda i: (0, _CAP_M // 128)),
            full(W1.shape), full((1, 128)),
            full(W2.shape), full((1, 64)),
            full(W3.shape), full((1, 32)),
            full(Wo.shape), full((1, 1)),
        ],
        out_specs=pl.BlockSpec((blk,), lambda i: (i,)),
        out_shape=jax.ShapeDtypeStruct((_B,), jnp.float32),
    )(xu, xm, u3, m3, tgu, tgm, tmu, tmm,
      W1, b1.reshape(1, 128), W2, b2.reshape(1, 64),
      W3, b3.reshape(1, 32), Wo, bo.reshape(1, 1))


def kernel(user, movie, gmf_user, gmf_movie, mlp_user, mlp_movie,
           W1, b1, W2, b2, W3, b3, Wo, bo):
    user = user.astype(jnp.int32)
    movie = movie.astype(jnp.int32)
    tgu, tgm = gmf_user.T, gmf_movie.T
    tmu, tmm = mlp_user.T, mlp_movie.T
    nch_u = (_STRIPE_U + _CHUNK - 1) // _CHUNK
    nch_m = (_STRIPE_M + _CHUNK - 1) // _CHUNK
    xu = _sc_side(user, tgu, tmu, _CAP_U, _STRIPE_U, nch_u)
    xm = _sc_side(movie, tgm, tmm, _CAP_M, _STRIPE_M, nch_m)
    return _tc_mlp(xu, xm,
                   user.reshape(_B, 1), movie.reshape(_B, 1),
                   tgu, tgm, tmu, tmm,
                   W1, b1, W2, b2, W3, b3, Wo, bo)


# speedup vs baseline: 1.1110x; 1.1110x over previous
import functools

import jax
import jax.numpy as jnp
from jax import lax
from jax.experimental import pallas as pl
from jax.experimental.pallas import tpu as pltpu
from jax.experimental.pallas import tpu_sc as plsc

_B = 16384
_D = 32


def _sc_gather(user, movie, gmf_user, gmf_movie, mlp_user, mlp_movie):
    info = plsc.get_sparse_core_info()
    nw = info.num_cores * info.num_subcores
    bpw = _B // nw
    mesh = plsc.VectorSubcoreMesh(core_axis_name="c", subcore_axis_name="s")

    @functools.partial(
        pl.kernel,
        mesh=mesh,
        compiler_params=pltpu.CompilerParams(use_tc_tiling_on_sc=False),
        out_type=[jax.ShapeDtypeStruct((_B, _D), jnp.float32)] * 4,
        scratch_types=[
            pltpu.VMEM((bpw,), jnp.int32),
            pltpu.VMEM((bpw,), jnp.int32),
            pltpu.VMEM((bpw, _D), jnp.float32),
            pltpu.VMEM((bpw, _D), jnp.float32),
            pltpu.VMEM((bpw, _D), jnp.float32),
            pltpu.VMEM((bpw, _D), jnp.float32),
            pltpu.SemaphoreType.DMA,
        ],
    )
    def k(user_h, movie_h, gu_h, gm_h, mu_h, mm_h,
          gu_o, gm_o, mu_o, mm_o,
          uidx, midx, gu_v, gm_v, mu_v, mm_v, sem):
        wid = lax.axis_index("s") * info.num_cores + lax.axis_index("c")
        base = wid * bpw
        pltpu.sync_copy(user_h.at[pl.ds(base, bpw)], uidx)
        pltpu.sync_copy(movie_h.at[pl.ds(base, bpw)], midx)
        c1 = pltpu.async_copy(gu_h.at[uidx], gu_v, sem)
        c2 = pltpu.async_copy(gm_h.at[midx], gm_v, sem)
        c3 = pltpu.async_copy(mu_h.at[uidx], mu_v, sem)
        c4 = pltpu.async_copy(mm_h.at[midx], mm_v, sem)
        c1.wait()
        c2.wait()
        c3.wait()
        c4.wait()
        pltpu.sync_copy(gu_v, gu_o.at[pl.ds(base, bpw)])
        pltpu.sync_copy(gm_v, gm_o.at[pl.ds(base, bpw)])
        pltpu.sync_copy(mu_v, mu_o.at[pl.ds(base, bpw)])
        pltpu.sync_copy(mm_v, mm_o.at[pl.ds(base, bpw)])

    return k(user, movie, gmf_user, gmf_movie, mlp_user, mlp_movie)


def _mlp_body(gu_r, gm_r, mu_r, mm_r, w1_r, b1_r, w2_r, b2_r, w3_r, b3_r,
              wo_r, bo_r, out_r):
    f32 = jnp.float32
    w1 = w1_r[...]
    h = (lax.dot_general(mu_r[...], w1[:, :_D], (((1,), (1,)), ((), ())),
                         preferred_element_type=f32)
         + lax.dot_general(mm_r[...], w1[:, _D:], (((1,), (1,)), ((), ())),
                           preferred_element_type=f32)
         + b1_r[...])
    h = jnp.maximum(h, 0.0)
    h = lax.dot_general(h, w2_r[...], (((1,), (1,)), ((), ())),
                        preferred_element_type=f32) + b2_r[...]
    h = jnp.maximum(h, 0.0)
    h = lax.dot_general(h, w3_r[...], (((1,), (1,)), ((), ())),
                        preferred_element_type=f32) + b3_r[...]
    h = jnp.maximum(h, 0.0)
    gmf = gu_r[...] * gm_r[...]
    wo = wo_r[...]
    out = (jnp.sum(gmf * wo[:, :_D], axis=1)
           + jnp.sum(h * wo[:, _D:], axis=1)
           + bo_r[0, 0])
    out_r[...] = out


def _tc_mlp(gu, gm, mu, mm, W1, b1, W2, b2, W3, b3, Wo, bo):
    blk = 2048
    grid = _B // blk
    row_spec = pl.BlockSpec((blk, _D), lambda i: (i, 0))

    def full(shape):
        return pl.BlockSpec(shape, lambda i: (0,) * len(shape))

    return pl.pallas_call(
        _mlp_body,
        grid=(grid,),
        in_specs=[
            row_spec, row_spec, row_spec, row_spec,
            full(W1.shape), full((1, 128)),
            full(W2.shape), full((1, 64)),
            full(W3.shape), full((1, 32)),
            full(Wo.shape), full((1, 1)),
        ],
        out_specs=pl.BlockSpec((blk,), lambda i: (i,)),
        out_shape=jax.ShapeDtypeStruct((_B,), jnp.float32),
    )(gu, gm, mu, mm, W1, b1.reshape(1, 128), W2, b2.reshape(1, 64),
      W3, b3.reshape(1, 32), Wo, bo.reshape(1, 1))


def kernel(user, movie, gmf_user, gmf_movie, mlp_user, mlp_movie,
           W1, b1, W2, b2, W3, b3, Wo, bo):
    user = user.astype(jnp.int32)
    movie = movie.astype(jnp.int32)
    gu, gm, mu, mm = _sc_gather(user, movie, gmf_user, gmf_movie,
                                mlp_user, mlp_movie)
    return _tc_mlp(gu, gm, mu, mm, W1, b1, W2, b2, W3, b3, Wo, bo)
